# baseline (speedup 1.0000x reference)
import functools

import jax
import jax.numpy as jnp
from jax import lax
from jax.experimental import pallas as pl
from jax.experimental.pallas import tpu as pltpu
from jax.experimental.pallas import tpu_sc as plsc

N = 50000
D = 128
E = 400000

NS = 16
B = 128
G = 28
NG = 7
HW = 64
CH = G * NG
EPT = CH * B
E_PAD = NS * EPT
NP = 50176
RPT = NP // NS
ZR = 112
DUMMY = N

TM = 400
GRID = N // TM


def _mesh():
    return plsc.VectorSubcoreMesh(core_axis_name="c", subcore_axis_name="s")


def _deg_body(zeros, srch0, dsth0, srch1, dsth1, ds0, dd0, ds1, dd1,
              accA, accB, idxA, idxB, ones):
    c = lax.axis_index("c")
    s = lax.axis_index("s")

    def fill_ones(i, _):
        ones[i, :] = jnp.full((16,), 1.0, jnp.float32)
        return 0
    lax.fori_loop(0, B, fill_ones, 0)

    zsl = pl.ds(pl.multiple_of(s * RPT, 8), RPT)
    pltpu.sync_copy(zeros.at[zsl], accA.at[zsl])
    pltpu.sync_copy(zeros.at[zsl], accB.at[zsl])
    plsc.subcore_barrier()

    def run(srch, dsth):
        def grp(g, _):
            pltpu.sync_copy(srch.at[s].at[g], idxA)
            pltpu.sync_copy(dsth.at[s].at[g], idxB)

            def step(j, _):
                pltpu.sync_copy(ones, accA.at[idxA.at[j]], add=True)
                pltpu.sync_copy(ones, accB.at[idxB.at[j]], add=True)
                return 0
            lax.fori_loop(0, G, step, 0)
            return 0
        lax.fori_loop(0, NG, grp, 0)

    @pl.when(c == 0)
    def _():
        run(srch0, dsth0)

    @pl.when(c == 1)
    def _():
        run(srch1, dsth1)

    plsc.subcore_barrier()
    sl = pl.ds(pl.multiple_of(s * RPT, 8), RPT)

    @pl.when(c == 0)
    def _():
        pltpu.sync_copy(accA.at[sl], ds0.at[sl])
        pltpu.sync_copy(accB.at[sl], dd0.at[sl])

    @pl.when(c == 1)
    def _():
        pltpu.sync_copy(accA.at[sl], ds1.at[sl])
        pltpu.sync_copy(accB.at[sl], dd1.at[sl])


def _deg_call(srch0, dsth0, srch1, dsth1):
    f = pl.kernel(
        _deg_body,
        out_type=[jax.ShapeDtypeStruct((NP, 16), jnp.float32)] * 4,
        mesh=_mesh(),
        scratch_types=[
            pltpu.VMEM_SHARED((NP, 16), jnp.float32),
            pltpu.VMEM_SHARED((NP, 16), jnp.float32),
            pltpu.VMEM((G, B), jnp.int32),
            pltpu.VMEM((G, B), jnp.int32),
            pltpu.VMEM((B, 16), jnp.float32),
        ],
        compiler_params=pltpu.CompilerParams(use_tc_tiling_on_sc=False),
    )
    return f(jnp.zeros((NP, 16), jnp.float32), srch0, dsth0, srch1, dsth1)


def _agg_body(zeros, ta0, ta1, tb0, tb1, srcg0, dsth0, srcg1, dsth1,
              oa0, oa1, ob0, ob1,
              acc, idxS, idxD, rows0, rows1, sem0, sem1):
    c = lax.axis_index("c")
    s = lax.axis_index("s")

    zsl = pl.ds(pl.multiple_of(s * RPT, 8), RPT)

    def run(tab, srcg, dsth):
        def grp(g, _):
            pltpu.sync_copy(srcg.at[s].at[g], idxS)
            pltpu.sync_copy(dsth.at[s].at[g], idxD)
            pltpu.async_copy(tab.at[idxS.at[0]], rows0, sem0)

            def pair(p, _):
                j = 2 * p
                pltpu.make_async_copy(tab.at[idxS.at[j]], rows0, sem0).wait()
                pltpu.async_copy(tab.at[idxS.at[j + 1]], rows1, sem1)
                pltpu.sync_copy(rows0, acc.at[idxD.at[j]], add=True)
                pltpu.make_async_copy(tab.at[idxS.at[j + 1]], rows1, sem1).wait()

                @pl.when(p + 1 < G // 2)
                def _():
                    pltpu.async_copy(tab.at[idxS.at[j + 2]], rows0, sem0)
                pltpu.sync_copy(rows1, acc.at[idxD.at[j + 1]], add=True)
                return 0
            lax.fori_loop(0, G // 2, pair, 0)
            return 0
        lax.fori_loop(0, NG, grp, 0)

    def half(tab0, tab1, out0, out1):
        pltpu.sync_copy(zeros.at[zsl], acc.at[zsl])
        plsc.subcore_barrier()

        @pl.when(c == 0)
        def _():
            run(tab0, srcg0, dsth0)

        @pl.when(c == 1)
        def _():
            run(tab1, srcg1, dsth1)

        plsc.subcore_barrier()

        @pl.when(c == 0)
        def _():
            pltpu.sync_copy(acc.at[zsl], out0.at[zsl])

        @pl.when(c == 1)
        def _():
            pltpu.sync_copy(acc.at[zsl], out1.at[zsl])

    half(ta0, ta1, oa0, oa1)
    half(tb0, tb1, ob0, ob1)


def _agg_call(ta0, ta1, tb0, tb1, srcg0, dsth0, srcg1, dsth1):
    f = pl.kernel(
        _agg_body,
        out_type=[jax.ShapeDtypeStruct((NP, HW), jnp.bfloat16)] * 4,
        mesh=_mesh(),
        scratch_types=[
            pltpu.VMEM_SHARED((NP, HW), jnp.bfloat16),
            pltpu.VMEM((G, B), jnp.int32),
            pltpu.VMEM((G, B), jnp.int32),
            pltpu.VMEM((B, HW), jnp.bfloat16),
            pltpu.VMEM((B, HW), jnp.bfloat16),
            pltpu.SemaphoreType.DMA,
            pltpu.SemaphoreType.DMA,
        ],
        compiler_params=pltpu.CompilerParams(use_tc_tiling_on_sc=False),
    )
    return f(jnp.zeros((NP, HW), jnp.bfloat16), ta0, ta1, tb0, tb1,
             srcg0, dsth0, srcg1, dsth1)


def _tc1_body(x_ref, w_ref, d0_ref, d1_ref, *outs):
    xw = jnp.dot(x_ref[...], w_ref[...], preferred_element_type=jnp.float32)
    ns0 = lax.rsqrt(jnp.maximum(d0_ref[...][:, :1], 1.0))
    ns1 = lax.rsqrt(jnp.maximum(d1_ref[...][:, :1], 1.0))
    y0 = (xw[:, :D] * ns0).astype(jnp.bfloat16)
    y1 = (xw[:, D:] * ns1).astype(jnp.bfloat16)
    for h in range(2):
        outs[h][...] = y0[:, HW * h:HW * h + HW]
        outs[2 + h][...] = y1[:, HW * h:HW * h + HW]


def _tc1(x, wc, ds0, ds1):
    return pl.pallas_call(
        _tc1_body,
        grid=(GRID,),
        in_specs=[
            pl.BlockSpec((TM, D), lambda i: (i, 0)),
            pl.BlockSpec((D, 2 * D), lambda i: (0, 0)),
            pl.BlockSpec((TM, 16), lambda i: (i, 0)),
            pl.BlockSpec((TM, 16), lambda i: (i, 0)),
        ],
        out_specs=[pl.BlockSpec((TM, HW), lambda i: (i, 0))] * 4,
        out_shape=[jax.ShapeDtypeStruct((N, HW), jnp.bfloat16)] * 4,
    )(x, wc, ds0, ds1)


def _tc2_body(a00, a01, a10, a11, dd0_ref, dd1_ref,
              b0_ref, b1_ref, w_ref, d0_ref, d1_ref, *outs):
    acc0 = jnp.concatenate([a00[...], a01[...]], axis=1).astype(jnp.float32)
    acc1 = jnp.concatenate([a10[...], a11[...]], axis=1).astype(jnp.float32)
    nd0 = lax.rsqrt(jnp.maximum(dd0_ref[...][:, :1], 1.0))
    nd1 = lax.rsqrt(jnp.maximum(dd1_ref[...][:, :1], 1.0))
    h = 0.5 * (acc0 * nd0 + b0_ref[...] + acc1 * nd1 + b1_ref[...])
    h = jnp.maximum(h, 0.0)
    hw = jnp.dot(h, w_ref[...], preferred_element_type=jnp.float32)
    ns0 = lax.rsqrt(jnp.maximum(d0_ref[...][:, :1], 1.0))
    ns1 = lax.rsqrt(jnp.maximum(d1_ref[...][:, :1], 1.0))
    y0 = (hw[:, :D] * ns0).astype(jnp.bfloat16)
    y1 = (hw[:, D:] * ns1).astype(jnp.bfloat16)
    for h2 in range(2):
        outs[h2][...] = y0[:, HW * h2:HW * h2 + HW]
        outs[2 + h2][...] = y1[:, HW * h2:HW * h2 + HW]


def _tc2(accs0, accs1, dd0, dd1, b0, b1, wc, ds0, ds1):
    hspec = pl.BlockSpec((TM, HW), lambda i: (i, 0))
    return pl.pallas_call(
        _tc2_body,
        grid=(GRID,),
        in_specs=[hspec] * 4 + [
            pl.BlockSpec((TM, 16), lambda i: (i, 0)),
            pl.BlockSpec((TM, 16), lambda i: (i, 0)),
            pl.BlockSpec((1, D), lambda i: (0, 0)),
            pl.BlockSpec((1, D), lambda i: (0, 0)),
            pl.BlockSpec((D, 2 * D), lambda i: (0, 0)),
            pl.BlockSpec((TM, 16), lambda i: (i, 0)),
            pl.BlockSpec((TM, 16), lambda i: (i, 0)),
        ],
        out_specs=[hspec] * 4,
        out_shape=[jax.ShapeDtypeStruct((N, HW), jnp.bfloat16)] * 4,
    )(*accs0, *accs1, dd0, dd1, b0, b1, wc, ds0, ds1)


def _tc3_body(a00, a01, a10, a11, dd0_ref, dd1_ref,
              b0_ref, b1_ref, wg_ref, bg_ref, out_ref):
    acc0 = jnp.concatenate([a00[...], a01[...]], axis=1).astype(jnp.float32)
    acc1 = jnp.concatenate([a10[...], a11[...]], axis=1).astype(jnp.float32)
    nd0 = lax.rsqrt(jnp.maximum(dd0_ref[...][:, :1], 1.0))
    nd1 = lax.rsqrt(jnp.maximum(dd1_ref[...][:, :1], 1.0))
    h = 0.5 * (acc0 * nd0 + b0_ref[...] + acc1 * nd1 + b1_ref[...])
    g = jnp.dot(h, wg_ref[...], preferred_element_type=jnp.float32) + bg_ref[...]
    Hh = D // 2

    def lstm(gs):
        ig = jax.nn.sigmoid(gs[:, 0 * Hh:1 * Hh])
        gg = jnp.tanh(gs[:, 2 * Hh:3 * Hh])
        og = jax.nn.sigmoid(gs[:, 3 * Hh:4 * Hh])
        return og * jnp.tanh(ig * gg)

    hf = lstm(g[:, :4 * Hh])
    hb = lstm(g[:, 4 * Hh:])
    out_ref[...] = jnp.concatenate([hf, hb], axis=1)


def _tc3(accs0, accs1, dd0, dd1, b0, b1, wg, bg):
    hspec = pl.BlockSpec((TM, HW), lambda i: (i, 0))
    return pl.pallas_call(
        _tc3_body,
        grid=(GRID,),
        in_specs=[hspec] * 4 + [
            pl.BlockSpec((TM, 16), lambda i: (i, 0)),
            pl.BlockSpec((TM, 16), lambda i: (i, 0)),
            pl.BlockSpec((1, D), lambda i: (0, 0)),
            pl.BlockSpec((1, D), lambda i: (0, 0)),
            pl.BlockSpec((D, 4 * D), lambda i: (0, 0)),
            pl.BlockSpec((1, 4 * D), lambda i: (0, 0)),
        ],
        out_specs=pl.BlockSpec((TM, D), lambda i: (i, 0)),
        out_shape=jax.ShapeDtypeStruct((N, D), jnp.float32),
    )(*accs0, *accs1, dd0, dd1, b0, b1, wg, bg)


def _prep_idx(ei):
    src = ei[0].astype(jnp.int32)
    dst = ei[1].astype(jnp.int32)
    pad = E_PAD - E
    srcg = jnp.concatenate([src, jnp.zeros((pad,), jnp.int32)]).reshape(NS, NG, G, B)
    srch = jnp.concatenate([src, jnp.full((pad,), DUMMY, jnp.int32)]).reshape(NS, NG, G, B)
    dsth = jnp.concatenate([dst, jnp.full((pad,), DUMMY, jnp.int32)]).reshape(NS, NG, G, B)
    return srcg, srch, dsth


def kernel(x_entity, W1_r0, b1_r0, W1_r1, b1_r1, W2_r0, b2_r0, W2_r1, b2_r1,
           W_ih_f, W_hh_f, b_ih_f, b_hh_f, W_ih_b, W_hh_b, b_ih_b, b_hh_b,
           edge_index_r0, edge_index_r1):
    srcg0, srch0, dsth0 = _prep_idx(edge_index_r0)
    srcg1, srch1, dsth1 = _prep_idx(edge_index_r1)

    wc1 = jnp.concatenate([W1_r0, W1_r1], axis=1)
    wc2 = jnp.concatenate([W2_r0, W2_r1], axis=1)
    wg = jnp.concatenate([W_ih_f.T, W_ih_b.T], axis=1)
    bg = jnp.concatenate([b_ih_f + b_hh_f, b_ih_b + b_hh_b]).reshape(1, 4 * D)
    b10 = b1_r0.reshape(1, D)
    b11 = b1_r1.reshape(1, D)
    b20 = b2_r0.reshape(1, D)
    b21 = b2_r1.reshape(1, D)

    ds0, dd0, ds1, dd1 = _deg_call(srch0, dsth0, srch1, dsth1)

    tabs = _tc1(x_entity, wc1, ds0, ds1)
    oa0, oa1, ob0, ob1 = _agg_call(tabs[0], tabs[2], tabs[1], tabs[3],
                                   srcg0, dsth0, srcg1, dsth1)
    accs0, accs1 = [oa0, ob0], [oa1, ob1]

    tabs2 = _tc2(accs0, accs1, dd0, dd1, b10, b11, wc2, ds0, ds1)
    oa0, oa1, ob0, ob1 = _agg_call(tabs2[0], tabs2[2], tabs2[1], tabs2[3],
                                   srcg0, dsth0, srcg1, dsth1)
    accs0, accs1 = [oa0, ob0], [oa1, ob1]

    return _tc3(accs0, accs1, dd0, dd1, b20, b21, wg, bg)


# speedup vs baseline: 1.1926x; 1.1926x over previous
import functools

import jax
import jax.numpy as jnp
from jax import lax
from jax.experimental import pallas as pl
from jax.experimental.pallas import tpu as pltpu
from jax.experimental.pallas import tpu_sc as plsc

N = 50000
D = 128
E = 400000

NS = 16
B = 128
HW = 64
G_D = 28
NG_D = 7
E_PAD_D = NS * NG_D * G_D * B
G_A = 18
NG_A = 11
E_PAD_A = NS * NG_A * G_A * B
NP = 50176
RPT = NP // NS
ZR = 112
DUMMY = N

TM = 400
GRID = N // TM


def _mesh():
    return plsc.VectorSubcoreMesh(core_axis_name="c", subcore_axis_name="s")


def _deg_body(zeros, srch0, dsth0, srch1, dsth1, ds0, dd0, ds1, dd1,
              accA, accB, idxA, idxB, ones):
    c = lax.axis_index("c")
    s = lax.axis_index("s")

    def fill_ones(i, _):
        ones[i, :] = jnp.full((16,), 1.0, jnp.float32)
        return 0
    lax.fori_loop(0, B, fill_ones, 0)

    zsl = pl.ds(pl.multiple_of(s * RPT, 8), RPT)
    pltpu.sync_copy(zeros.at[zsl], accA.at[zsl])
    pltpu.sync_copy(zeros.at[zsl], accB.at[zsl])
    plsc.subcore_barrier()

    def run(srch, dsth):
        def grp(g, _):
            pltpu.sync_copy(srch.at[s].at[g], idxA)
            pltpu.sync_copy(dsth.at[s].at[g], idxB)

            def step(j, _):
                pltpu.sync_copy(ones, accA.at[idxA.at[j]], add=True)
                pltpu.sync_copy(ones, accB.at[idxB.at[j]], add=True)
                return 0
            lax.fori_loop(0, G_D, step, 0)
            return 0
        lax.fori_loop(0, NG_D, grp, 0)

    @pl.when(c == 0)
    def _():
        run(srch0, dsth0)

    @pl.when(c == 1)
    def _():
        run(srch1, dsth1)

    plsc.subcore_barrier()
    sl = pl.ds(pl.multiple_of(s * RPT, 8), RPT)

    @pl.when(c == 0)
    def _():
        pltpu.sync_copy(accA.at[sl], ds0.at[sl])
        pltpu.sync_copy(accB.at[sl], dd0.at[sl])

    @pl.when(c == 1)
    def _():
        pltpu.sync_copy(accA.at[sl], ds1.at[sl])
        pltpu.sync_copy(accB.at[sl], dd1.at[sl])


def _deg_call(srch0, dsth0, srch1, dsth1):
    f = pl.kernel(
        _deg_body,
        out_type=[jax.ShapeDtypeStruct((NP, 16), jnp.float32)] * 4,
        mesh=_mesh(),
        scratch_types=[
            pltpu.VMEM_SHARED((NP, 16), jnp.float32),
            pltpu.VMEM_SHARED((NP, 16), jnp.float32),
            pltpu.VMEM((G_D, B), jnp.int32),
            pltpu.VMEM((G_D, B), jnp.int32),
            pltpu.VMEM((B, 16), jnp.float32),
        ],
        compiler_params=pltpu.CompilerParams(use_tc_tiling_on_sc=False),
    )
    return f(jnp.zeros((NP, 16), jnp.float32), srch0, dsth0, srch1, dsth1)


def _agg_body(zeros, tab0, tab1, srcg0, dsth0, srcg1, dsth1, out0, out1,
              acc, idxS, idxD, r0, r1, r2,
              gs0, gs1, gs2, ss0, ss1, ss2):
    c = lax.axis_index("c")
    s = lax.axis_index("s")
    rows = [r0, r1, r2]
    gsem = [gs0, gs1, gs2]
    ssem = [ss0, ss1, ss2]

    zsl = pl.ds(pl.multiple_of(s * RPT, 8), RPT)
    pltpu.sync_copy(zeros.at[zsl], acc.at[zsl])
    plsc.subcore_barrier()

    def run(tab, srcg, dsth):
        def grp(g, _):
            pltpu.sync_copy(srcg.at[s].at[g], idxS)
            pltpu.sync_copy(dsth.at[s].at[g], idxD)

            def triad(q, _):
                for b in range(3):
                    j = 3 * q + b

                    @pl.when(q > 0)
                    def _():
                        pltpu.make_async_copy(
                            rows[b], acc.at[idxD.at[j]], ssem[b]).wait()
                    pltpu.async_copy(tab.at[idxS.at[j]], rows[b], gsem[b])
                for b in range(3):
                    j = 3 * q + b
                    pltpu.make_async_copy(
                        tab.at[idxS.at[j]], rows[b], gsem[b]).wait()
                    pltpu.async_copy(
                        rows[b], acc.at[idxD.at[j]], ssem[b], add=True)
                return 0
            lax.fori_loop(0, G_A // 3, triad, 0)
            for b in range(3):
                pltpu.make_async_copy(rows[b], acc.at[idxD.at[b]], ssem[b]).wait()
            return 0
        lax.fori_loop(0, NG_A, grp, 0)

    @pl.when(c == 0)
    def _():
        run(tab0, srcg0, dsth0)

    @pl.when(c == 1)
    def _():
        run(tab1, srcg1, dsth1)

    plsc.subcore_barrier()

    @pl.when(c == 0)
    def _():
        pltpu.sync_copy(acc.at[zsl], out0.at[zsl])

    @pl.when(c == 1)
    def _():
        pltpu.sync_copy(acc.at[zsl], out1.at[zsl])


def _agg_call(tab0, tab1, srcg0, dsth0, srcg1, dsth1):
    f = pl.kernel(
        _agg_body,
        out_type=[jax.ShapeDtypeStruct((NP, HW), jnp.bfloat16)] * 2,
        mesh=_mesh(),
        scratch_types=[
            pltpu.VMEM_SHARED((NP, HW), jnp.bfloat16),
            pltpu.VMEM((G_A, B), jnp.int32),
            pltpu.VMEM((G_A, B), jnp.int32),
        ] + [pltpu.VMEM((B, HW), jnp.bfloat16)] * 3
          + [pltpu.SemaphoreType.DMA] * 6,
        compiler_params=pltpu.CompilerParams(use_tc_tiling_on_sc=False),
    )
    return f(jnp.zeros((NP, HW), jnp.bfloat16), tab0, tab1,
             srcg0, dsth0, srcg1, dsth1)


def _tc1_body(x_ref, w_ref, d0_ref, d1_ref, *outs):
    xw = jnp.dot(x_ref[...], w_ref[...], preferred_element_type=jnp.float32)
    ns0 = lax.rsqrt(jnp.maximum(d0_ref[...][:, :1], 1.0))
    ns1 = lax.rsqrt(jnp.maximum(d1_ref[...][:, :1], 1.0))
    y0 = (xw[:, :D] * ns0).astype(jnp.bfloat16)
    y1 = (xw[:, D:] * ns1).astype(jnp.bfloat16)
    for h in range(2):
        outs[h][...] = y0[:, HW * h:HW * h + HW]
        outs[2 + h][...] = y1[:, HW * h:HW * h + HW]


def _tc1(x, wc, ds0, ds1):
    return pl.pallas_call(
        _tc1_body,
        grid=(GRID,),
        in_specs=[
            pl.BlockSpec((TM, D), lambda i: (i, 0)),
            pl.BlockSpec((D, 2 * D), lambda i: (0, 0)),
            pl.BlockSpec((TM, 16), lambda i: (i, 0)),
            pl.BlockSpec((TM, 16), lambda i: (i, 0)),
        ],
        out_specs=[pl.BlockSpec((TM, HW), lambda i: (i, 0))] * 4,
        out_shape=[jax.ShapeDtypeStruct((N, HW), jnp.bfloat16)] * 4,
    )(x, wc, ds0, ds1)


def _tc2_body(a00, a01, a10, a11, dd0_ref, dd1_ref,
              b0_ref, b1_ref, w_ref, d0_ref, d1_ref, *outs):
    acc0 = jnp.concatenate([a00[...], a01[...]], axis=1).astype(jnp.float32)
    acc1 = jnp.concatenate([a10[...], a11[...]], axis=1).astype(jnp.float32)
    nd0 = lax.rsqrt(jnp.maximum(dd0_ref[...][:, :1], 1.0))
    nd1 = lax.rsqrt(jnp.maximum(dd1_ref[...][:, :1], 1.0))
    h = 0.5 * (acc0 * nd0 + b0_ref[...] + acc1 * nd1 + b1_ref[...])
    h = jnp.maximum(h, 0.0)
    hw = jnp.dot(h, w_ref[...], preferred_element_type=jnp.float32)
    ns0 = lax.rsqrt(jnp.maximum(d0_ref[...][:, :1], 1.0))
    ns1 = lax.rsqrt(jnp.maximum(d1_ref[...][:, :1], 1.0))
    y0 = (hw[:, :D] * ns0).astype(jnp.bfloat16)
    y1 = (hw[:, D:] * ns1).astype(jnp.bfloat16)
    for h2 in range(2):
        outs[h2][...] = y0[:, HW * h2:HW * h2 + HW]
        outs[2 + h2][...] = y1[:, HW * h2:HW * h2 + HW]


def _tc2(accs0, accs1, dd0, dd1, b0, b1, wc, ds0, ds1):
    hspec = pl.BlockSpec((TM, HW), lambda i: (i, 0))
    return pl.pallas_call(
        _tc2_body,
        grid=(GRID,),
        in_specs=[hspec] * 4 + [
            pl.BlockSpec((TM, 16), lambda i: (i, 0)),
            pl.BlockSpec((TM, 16), lambda i: (i, 0)),
            pl.BlockSpec((1, D), lambda i: (0, 0)),
            pl.BlockSpec((1, D), lambda i: (0, 0)),
            pl.BlockSpec((D, 2 * D), lambda i: (0, 0)),
            pl.BlockSpec((TM, 16), lambda i: (i, 0)),
            pl.BlockSpec((TM, 16), lambda i: (i, 0)),
        ],
        out_specs=[hspec] * 4,
        out_shape=[jax.ShapeDtypeStruct((N, HW), jnp.bfloat16)] * 4,
    )(*accs0, *accs1, dd0, dd1, b0, b1, wc, ds0, ds1)


def _tc3_body(a00, a01, a10, a11, dd0_ref, dd1_ref,
              b0_ref, b1_ref, wg_ref, bg_ref, out_ref):
    acc0 = jnp.concatenate([a00[...], a01[...]], axis=1).astype(jnp.float32)
    acc1 = jnp.concatenate([a10[...], a11[...]], axis=1).astype(jnp.float32)
    nd0 = lax.rsqrt(jnp.maximum(dd0_ref[...][:, :1], 1.0))
    nd1 = lax.rsqrt(jnp.maximum(dd1_ref[...][:, :1], 1.0))
    h = 0.5 * (acc0 * nd0 + b0_ref[...] + acc1 * nd1 + b1_ref[...])
    g = jnp.dot(h, wg_ref[...], preferred_element_type=jnp.float32) + bg_ref[...]
    Hh = D // 2

    def lstm(gs):
        ig = jax.nn.sigmoid(gs[:, 0 * Hh:1 * Hh])
        gg = jnp.tanh(gs[:, 2 * Hh:3 * Hh])
        og = jax.nn.sigmoid(gs[:, 3 * Hh:4 * Hh])
        return og * jnp.tanh(ig * gg)

    hf = lstm(g[:, :4 * Hh])
    hb = lstm(g[:, 4 * Hh:])
    out_ref[...] = jnp.concatenate([hf, hb], axis=1)


def _tc3(accs0, accs1, dd0, dd1, b0, b1, wg, bg):
    hspec = pl.BlockSpec((TM, HW), lambda i: (i, 0))
    return pl.pallas_call(
        _tc3_body,
        grid=(GRID,),
        in_specs=[hspec] * 4 + [
            pl.BlockSpec((TM, 16), lambda i: (i, 0)),
            pl.BlockSpec((TM, 16), lambda i: (i, 0)),
            pl.BlockSpec((1, D), lambda i: (0, 0)),
            pl.BlockSpec((1, D), lambda i: (0, 0)),
            pl.BlockSpec((D, 4 * D), lambda i: (0, 0)),
            pl.BlockSpec((1, 4 * D), lambda i: (0, 0)),
        ],
        out_specs=pl.BlockSpec((TM, D), lambda i: (i, 0)),
        out_shape=jax.ShapeDtypeStruct((N, D), jnp.float32),
    )(*accs0, *accs1, dd0, dd1, b0, b1, wg, bg)


def _prep_idx(ei):
    src = ei[0].astype(jnp.int32)
    dst = ei[1].astype(jnp.int32)
    pad_a = E_PAD_A - E
    pad_d = E_PAD_D - E
    srcg = jnp.concatenate([src, jnp.zeros((pad_a,), jnp.int32)]).reshape(
        NS, NG_A, G_A, B)
    dstg = jnp.concatenate([dst, jnp.full((pad_a,), DUMMY, jnp.int32)]).reshape(
        NS, NG_A, G_A, B)
    srch = jnp.concatenate([src, jnp.full((pad_d,), DUMMY, jnp.int32)]).reshape(
        NS, NG_D, G_D, B)
    dsth = jnp.concatenate([dst, jnp.full((pad_d,), DUMMY, jnp.int32)]).reshape(
        NS, NG_D, G_D, B)
    return srcg, dstg, srch, dsth


def kernel(x_entity, W1_r0, b1_r0, W1_r1, b1_r1, W2_r0, b2_r0, W2_r1, b2_r1,
           W_ih_f, W_hh_f, b_ih_f, b_hh_f, W_ih_b, W_hh_b, b_ih_b, b_hh_b,
           edge_index_r0, edge_index_r1):
    srcg0, dstg0, srch0, dsth0 = _prep_idx(edge_index_r0)
    srcg1, dstg1, srch1, dsth1 = _prep_idx(edge_index_r1)

    wc1 = jnp.concatenate([W1_r0, W1_r1], axis=1)
    wc2 = jnp.concatenate([W2_r0, W2_r1], axis=1)
    wg = jnp.concatenate([W_ih_f.T, W_ih_b.T], axis=1)
    bg = jnp.concatenate([b_ih_f + b_hh_f, b_ih_b + b_hh_b]).reshape(1, 4 * D)
    b10 = b1_r0.reshape(1, D)
    b11 = b1_r1.reshape(1, D)
    b20 = b2_r0.reshape(1, D)
    b21 = b2_r1.reshape(1, D)

    ds0, dd0, ds1, dd1 = _deg_call(srch0, dsth0, srch1, dsth1)

    tabs = _tc1(x_entity, wc1, ds0, ds1)
    accs0, accs1 = [], []
    for h in range(2):
        a0, a1 = _agg_call(tabs[h], tabs[2 + h], srcg0, dstg0, srcg1, dstg1)
        accs0.append(a0)
        accs1.append(a1)

    tabs2 = _tc2(accs0, accs1, dd0, dd1, b10, b11, wc2, ds0, ds1)
    accs0, accs1 = [], []
    for h in range(2):
        a0, a1 = _agg_call(tabs2[h], tabs2[2 + h], srcg0, dstg0, srcg1, dstg1)
        accs0.append(a0)
        accs1.append(a1)

    return _tc3(accs0, accs1, dd0, dd1, b20, b21, wg, bg)
